# baseline (speedup 1.0000x reference)
import functools

import jax
import jax.numpy as jnp
from jax import lax
from jax.experimental import pallas as pl
from jax.experimental.pallas import tpu as pltpu
from jax.experimental.pallas import tpu_sc as plsc

_SC_BATCHES = 0
_W = 384
_CHROWS = 24
_NW = 32
_LANES = 16

_LOG2_POLY = (
    -3.028317481039271,
    6.065830143185771,
    -5.2641104770847,
    3.2188328370634505,
    -1.2342631730389073,
    0.2668588228611466,
    -0.024825606614389147,
)
_LN2 = 0.6931471805599453


def _focal_terms_tanh(x, t):
    pos = t >= 1.0
    th = jnp.tanh(0.5 * x)
    s = jnp.where(pos, th, -th)
    arg = jnp.clip(0.5 + 0.5 * s, 1e-4, 1.0 - 1e-4)
    oma = 1.0 - arg
    omt = 1.0 - t
    omt2 = omt * omt
    w = (oma * oma) * jnp.where(pos, 1.0, omt2 * omt2)
    return arg, w, pos


def _focal_terms(x, t):
    p = jnp.clip(1.0 / (1.0 + jnp.exp(-x)), 1e-4, 1.0 - 1e-4)
    pos = t >= 1.0
    arg = jnp.where(pos, p, 1.0 - p)
    omp = 1.0 - p
    omt = 1.0 - t
    omt2 = omt * omt
    w = jnp.where(pos, omp * omp, (p * p) * (omt2 * omt2))
    return arg, w, pos


_TC_SH = 16


def _tc_body(x_ref, t_ref, out_ref, acc_ref):
    i = pl.program_id(0)
    blk, c, h, lanes = x_ref.shape
    sph = h // _TC_SH
    zero = jnp.zeros((_TC_SH, lanes), jnp.float32)

    def step(idx, carry):
        acc, cnt = carry
        for u in range(2):
            id2 = idx * 2 + u
            b = id2 // sph
            r = id2 % sph
            for ch in range(c):
                xs = x_ref[b, ch, pl.ds(r * _TC_SH, _TC_SH), :]
                ts = t_ref[b, ch, pl.ds(r * _TC_SH, _TC_SH), :]
                arg, w, pos = _focal_terms_tanh(xs, ts)
                acc = acc + jnp.log2(arg) * w
                cnt = cnt + jnp.where(pos, 1.0, 0.0)
        return acc, cnt

    part8, cnt8 = lax.fori_loop(0, blk * sph // 2, step, (zero, zero))
    part = jnp.sum(part8.reshape(_TC_SH // 8, 8, lanes), axis=0)
    cnt = jnp.sum(cnt8.reshape(_TC_SH // 8, 8, lanes), axis=0)

    @pl.when(i == 0)
    def _init():
        acc_ref[0] = part
        acc_ref[1] = cnt

    @pl.when(i > 0)
    def _acc():
        acc_ref[0] += part
        acc_ref[1] += cnt

    @pl.when(i == pl.num_programs(0) - 1)
    def _fin():
        loss = jnp.sum(acc_ref[0]) * jnp.float32(_LN2)
        npos = jnp.maximum(jnp.sum(acc_ref[1]), 1.0)
        out_ref[0] = -loss / npos


def _tc_loss(x4, t4, grid, b0):
    b, c, h, w = x4.shape
    blk = (b - b0) // grid
    off = b0 // blk
    out = pl.pallas_call(
        _tc_body,
        grid=(grid,),
        in_specs=[
            pl.BlockSpec((blk, c, h, w), lambda i: (i + off, 0, 0, 0)),
            pl.BlockSpec((blk, c, h, w), lambda i: (i + off, 0, 0, 0)),
        ],
        out_specs=pl.BlockSpec(memory_space=pltpu.SMEM),
        out_shape=jax.ShapeDtypeStruct((1,), jnp.float32),
        scratch_shapes=[pltpu.VMEM((2, 8, w), jnp.float32)],
    )(x4, t4)
    return out[0]


def _sc_log(arg):
    bits = lax.bitcast_convert_type(arg, jnp.int32)
    e = lax.convert_element_type(
        lax.shift_right_arithmetic(bits, 23) - 127, jnp.float32
    )
    m = lax.bitcast_convert_type(
        lax.bitwise_or(lax.bitwise_and(bits, 0x007FFFFF), 0x3F800000),
        jnp.float32,
    )
    acc = jnp.full((_LANES,), _LOG2_POLY[6], jnp.float32)
    for k in range(5, -1, -1):
        acc = acc * m + jnp.float32(_LOG2_POLY[k])
    return (e + acc) * jnp.float32(_LN2)


def _sc_body(nchunks, x_hbm, t_hbm, out_hbm, xbuf, tbuf, stage, sem):
    wid = lax.axis_index("s") * 2 + lax.axis_index("c")
    base = wid * (nchunks * _CHROWS)

    def chunk_step(c, carry):
        acc, cnt = carry
        row0 = base + c * _CHROWS
        pltpu.sync_copy(x_hbm.at[pl.ds(row0, _CHROWS)], xbuf)
        pltpu.sync_copy(t_hbm.at[pl.ds(row0, _CHROWS)], tbuf)

        def row_step(r, carry2):
            def vec_step(v, carry3):
                acc3, cnt3 = carry3
                xv = xbuf[r, pl.ds(v * _LANES, _LANES)]
                tv = tbuf[r, pl.ds(v * _LANES, _LANES)]
                arg, w, pos = _focal_terms(xv, tv)
                acc3 = acc3 + _sc_log(arg) * w
                cnt3 = cnt3 + jnp.where(pos, 1.0, 0.0)
                return acc3, cnt3

            return lax.fori_loop(0, _W // _LANES, vec_step, carry2)

        return lax.fori_loop(0, _CHROWS, row_step, (acc, cnt))

    zero = jnp.zeros((_LANES,), jnp.float32)
    acc, cnt = lax.fori_loop(0, nchunks, chunk_step, (zero, zero))
    stage[0, :] = acc
    stage[1, :] = cnt
    pltpu.sync_copy(stage, out_hbm.at[wid])


def _sc_sums(x2, t2, nchunks):
    mesh = plsc.VectorSubcoreMesh(core_axis_name="c", subcore_axis_name="s")
    kern = functools.partial(
        pl.kernel,
        mesh=mesh,
        out_type=jax.ShapeDtypeStruct((_NW, 2, _LANES), jnp.float32),
        scratch_types=[
            pltpu.VMEM((_CHROWS, _W), jnp.float32),
            pltpu.VMEM((_CHROWS, _W), jnp.float32),
            pltpu.VMEM((2, _LANES), jnp.float32),
            pltpu.SemaphoreType.DMA,
        ],
    )(functools.partial(_sc_body, nchunks))
    return kern(x2, t2)


def kernel(main_kf_logits, heatmap_target):
    b, c, h, w = main_kf_logits.shape
    grid = 2
    return _tc_loss(main_kf_logits, heatmap_target, grid, 0)


# speedup vs baseline: 1.0956x; 1.0956x over previous
import functools

import jax
import jax.numpy as jnp
from jax import lax
from jax.experimental import pallas as pl
from jax.experimental.pallas import tpu as pltpu
from jax.experimental.pallas import tpu_sc as plsc

_W = 384
_CHROWS = 24
_NW = 32
_LANES = 16

_LOG2_POLY = (
    -3.028317481039271,
    6.065830143185771,
    -5.2641104770847,
    3.2188328370634505,
    -1.2342631730389073,
    0.2668588228611466,
    -0.024825606614389147,
)
_LN2 = 0.6931471805599453


def _focal_terms_tanh(x, t):
    pos = t >= 1.0
    th = jnp.tanh(0.5 * x)
    s = jnp.where(pos, th, -th)
    arg = jnp.clip(0.5 + 0.5 * s, 1e-4, 1.0 - 1e-4)
    oma = 1.0 - arg
    omt = 1.0 - t
    omt2 = omt * omt
    w = (oma * oma) * jnp.where(pos, 1.0, omt2 * omt2)
    return arg, w, pos


def _focal_terms(x, t):
    p = jnp.clip(1.0 / (1.0 + jnp.exp(-x)), 1e-4, 1.0 - 1e-4)
    pos = t >= 1.0
    arg = jnp.where(pos, p, 1.0 - p)
    omp = 1.0 - p
    omt = 1.0 - t
    omt2 = omt * omt
    w = jnp.where(pos, omp * omp, (p * p) * (omt2 * omt2))
    return arg, w, pos


_TC_SH = 16


def _tc_body(x_ref, t_ref, out_ref, acc_ref):
    i = pl.program_id(0)
    blk, c, h, lanes = x_ref.shape
    sph = h // _TC_SH
    zero = jnp.zeros((_TC_SH, lanes), jnp.float32)

    def step(idx, carry):
        acc, cnt = carry
        for u in range(2):
            id2 = idx * 2 + u
            b = id2 // sph
            r = id2 % sph
            for ch in range(c):
                xs = x_ref[b, ch, pl.ds(r * _TC_SH, _TC_SH), :]
                ts = t_ref[b, ch, pl.ds(r * _TC_SH, _TC_SH), :]
                arg, w, pos = _focal_terms_tanh(xs, ts)
                acc = acc + jnp.log2(arg) * w
                cnt = cnt + jnp.where(pos, 1.0, 0.0)
        return acc, cnt

    part8, cnt8 = lax.fori_loop(0, blk * sph // 2, step, (zero, zero))
    part = jnp.sum(part8.reshape(_TC_SH // 8, 8, lanes), axis=0)
    cnt = jnp.sum(cnt8.reshape(_TC_SH // 8, 8, lanes), axis=0)

    @pl.when(i == 0)
    def _init():
        acc_ref[0] = part
        acc_ref[1] = cnt

    @pl.when(i > 0)
    def _acc():
        acc_ref[0] += part
        acc_ref[1] += cnt

    @pl.when(i == pl.num_programs(0) - 1)
    def _fin():
        loss = jnp.sum(acc_ref[0]) * jnp.float32(_LN2)
        npos = jnp.maximum(jnp.sum(acc_ref[1]), 1.0)
        out_ref[0] = -loss / npos


def _tc_loss(x4, t4, grid, b0):
    b, c, h, w = x4.shape
    blk = (b - b0) // grid
    off = b0 // blk
    out = pl.pallas_call(
        _tc_body,
        grid=(grid,),
        in_specs=[
            pl.BlockSpec((blk, c, h, w), lambda i: (i + off, 0, 0, 0)),
            pl.BlockSpec((blk, c, h, w), lambda i: (i + off, 0, 0, 0)),
        ],
        out_specs=pl.BlockSpec(memory_space=pltpu.SMEM),
        out_shape=jax.ShapeDtypeStruct((1,), jnp.float32),
        scratch_shapes=[pltpu.VMEM((2, 8, w), jnp.float32)],
    )(x4, t4)
    return out[0]


def _sc_log(arg):
    bits = lax.bitcast_convert_type(arg, jnp.int32)
    e = lax.convert_element_type(
        lax.shift_right_arithmetic(bits, 23) - 127, jnp.float32
    )
    m = lax.bitcast_convert_type(
        lax.bitwise_or(lax.bitwise_and(bits, 0x007FFFFF), 0x3F800000),
        jnp.float32,
    )
    acc = jnp.full((_LANES,), _LOG2_POLY[6], jnp.float32)
    for k in range(5, -1, -1):
        acc = acc * m + jnp.float32(_LOG2_POLY[k])
    return (e + acc) * jnp.float32(_LN2)


def _sc_body(nchunks, x_hbm, t_hbm, out_hbm, xbuf, tbuf, stage, sem):
    wid = lax.axis_index("s") * 2 + lax.axis_index("c")
    base = wid * (nchunks * _CHROWS)

    def chunk_step(c, carry):
        acc, cnt = carry
        row0 = base + c * _CHROWS
        pltpu.sync_copy(x_hbm.at[pl.ds(row0, _CHROWS)], xbuf)
        pltpu.sync_copy(t_hbm.at[pl.ds(row0, _CHROWS)], tbuf)

        def row_step(r, carry2):
            def vec_step(v, carry3):
                acc3, cnt3 = carry3
                xv = xbuf[r, pl.ds(v * _LANES, _LANES)]
                tv = tbuf[r, pl.ds(v * _LANES, _LANES)]
                arg, w, pos = _focal_terms(xv, tv)
                acc3 = acc3 + _sc_log(arg) * w
                cnt3 = cnt3 + jnp.where(pos, 1.0, 0.0)
                return acc3, cnt3

            return lax.fori_loop(0, _W // _LANES, vec_step, carry2)

        return lax.fori_loop(0, _CHROWS, row_step, (acc, cnt))

    zero = jnp.zeros((_LANES,), jnp.float32)
    acc, cnt = lax.fori_loop(0, nchunks, chunk_step, (zero, zero))
    stage[0, :] = acc
    stage[1, :] = cnt
    pltpu.sync_copy(stage, out_hbm.at[wid])


def _sc_sums(x2, t2, nchunks):
    mesh = plsc.VectorSubcoreMesh(core_axis_name="c", subcore_axis_name="s")
    kern = functools.partial(
        pl.kernel,
        mesh=mesh,
        out_type=jax.ShapeDtypeStruct((_NW, 2, _LANES), jnp.float32),
        scratch_types=[
            pltpu.VMEM((_CHROWS, _W), jnp.float32),
            pltpu.VMEM((_CHROWS, _W), jnp.float32),
            pltpu.VMEM((2, _LANES), jnp.float32),
            pltpu.SemaphoreType.DMA,
        ],
    )(functools.partial(_sc_body, nchunks))
    return kern(x2, t2)


def kernel(main_kf_logits, heatmap_target):
    return _tc_loss(main_kf_logits, heatmap_target, grid=4, b0=0)
